# baseline (speedup 1.0000x reference)
import jax
import jax.numpy as jnp
from jax.experimental import pallas as pl
from jax.experimental.pallas import tpu as pltpu

_LORA_SCALE = 0.2 * (32 / 16)
_R = 16


def _dot_t(a, b):
    return jax.lax.dot_general(a, b, (((1,), (1,)), ((), ())),
                               preferred_element_type=jnp.float32)


def _fused_body(uid_ref, x_ref, w_ref, b_ref, wcat_ref, wru_ref, uemb_ref,
                bcat_ref, o_ref, xbf_ref, v_ref):
    n = pl.program_id(1)

    @pl.when(n == 0)
    def _router():
        xbf = x_ref[...].astype(jnp.bfloat16)
        xbf_ref[...] = xbf
        r = _dot_t(xbf, wcat_ref[...])
        lu = _dot_t(uemb_ref[0], wru_ref[...])
        pi0 = jax.nn.sigmoid(r[:, 32:33] - r[:, 33:34])
        pu0 = jax.nn.sigmoid(lu[:, 0:1] - lu[:, 1:2])
        pm0 = jax.nn.sigmoid(2.0 * (pi0 + pu0) - 2.0)
        v_ref[:, :_R] = (r[:, :_R] * (_LORA_SCALE * pm0)).astype(jnp.bfloat16)
        v_ref[:, _R:] = (r[:, _R:2 * _R] * (_LORA_SCALE * (1.0 - pm0))).astype(jnp.bfloat16)

    acc = _dot_t(xbf_ref[...], w_ref[...])
    acc += _dot_t(v_ref[...], bcat_ref[...])
    o_ref[...] = acc + b_ref[...]


def kernel(x, user_id, W, b, W_route_in, W_route_user, user_emb, A0, B0, A1, B1):
    Bb, S, D_in = x.shape
    D_out = W.shape[0]
    M = Bb * S
    TM, TN = 1024, 512
    bf16 = jnp.bfloat16
    x2 = x.reshape(M, D_in)
    b2 = b.reshape(1, D_out)
    uid = (user_id[0].astype(jnp.int32) - 1)[None]
    W = W.astype(bf16)
    Wcat = jnp.concatenate([A0, A1, W_route_in], axis=0).astype(bf16)
    W_route_user = W_route_user.astype(bf16)
    user_emb = user_emb.astype(bf16).reshape(user_emb.shape[0], 1, D_in)
    Bcat = jnp.concatenate([B0, B1], axis=1).astype(bf16)

    grid = (M // TM, D_out // TN)
    out = pl.pallas_call(
        _fused_body,
        grid_spec=pltpu.PrefetchScalarGridSpec(
            num_scalar_prefetch=1,
            grid=grid,
            in_specs=[
                pl.BlockSpec((TM, D_in), lambda m, n, u: (m, 0)),
                pl.BlockSpec((TN, D_in), lambda m, n, u: (n, 0)),
                pl.BlockSpec((1, TN), lambda m, n, u: (0, n)),
                pl.BlockSpec((34, D_in), lambda m, n, u: (0, 0)),
                pl.BlockSpec((2, D_in), lambda m, n, u: (0, 0)),
                pl.BlockSpec((1, 1, D_in), lambda m, n, u: (u[0], 0, 0)),
                pl.BlockSpec((TN, 2 * _R), lambda m, n, u: (n, 0)),
            ],
            out_specs=pl.BlockSpec((TM, TN), lambda m, n, u: (m, n)),
            scratch_shapes=[pltpu.VMEM((TM, D_in), bf16),
                            pltpu.VMEM((TM, 2 * _R), bf16)],
        ),
        out_shape=jax.ShapeDtypeStruct((M, D_out), jnp.float32),
        compiler_params=pltpu.CompilerParams(
            dimension_semantics=("parallel", "arbitrary"),
        ),
    )(uid, x2, W, b2, Wcat, W_route_user, user_emb, Bcat)
    return out.reshape(Bb, S, D_out)


# speedup vs baseline: 1.0138x; 1.0138x over previous
import jax
import jax.numpy as jnp
from jax.experimental import pallas as pl
from jax.experimental.pallas import tpu as pltpu

_LORA_SCALE = 0.2 * (32 / 16)
_R = 16


def _dot_t(a, b):
    return jax.lax.dot_general(a, b, (((1,), (1,)), ((), ())),
                               preferred_element_type=jnp.float32)


def _fused_body(uid_ref, x_ref, w_ref, b_ref, wcat_ref, wru_ref, uemb_ref,
                bcat_ref, o_ref, v_ref):
    n = pl.program_id(1)

    @pl.when(n == 0)
    def _router():
        r = _dot_t(x_ref[...], wcat_ref[...])
        lu = _dot_t(uemb_ref[0], wru_ref[...])
        pi0 = jax.nn.sigmoid(r[:, 32:33] - r[:, 33:34])
        pu0 = jax.nn.sigmoid(lu[:, 0:1] - lu[:, 1:2])
        pm0 = jax.nn.sigmoid(2.0 * (pi0 + pu0) - 2.0)
        v_ref[:, :_R] = r[:, :_R] * (_LORA_SCALE * pm0)
        v_ref[:, _R:] = r[:, _R:2 * _R] * (_LORA_SCALE * (1.0 - pm0))

    acc = _dot_t(x_ref[...], w_ref[...])
    acc += _dot_t(v_ref[...], bcat_ref[...])
    o_ref[...] = acc + b_ref[...]


def kernel(x, user_id, W, b, W_route_in, W_route_user, user_emb, A0, B0, A1, B1):
    Bb, S, D_in = x.shape
    D_out = W.shape[0]
    M = Bb * S
    TM, TN = 1024, 512
    bf16 = jnp.bfloat16
    x2 = x.reshape(M, D_in)
    b2 = b.reshape(1, D_out)
    uid = (user_id[0].astype(jnp.int32) - 1)[None]
    W = W.astype(bf16)
    Wcat = jnp.concatenate([A0, A1, W_route_in], axis=0).astype(bf16)
    user_emb = user_emb.reshape(user_emb.shape[0], 1, D_in)
    Bcat = jnp.concatenate([B0, B1], axis=1).astype(bf16)

    grid = (M // TM, D_out // TN)
    out = pl.pallas_call(
        _fused_body,
        grid_spec=pltpu.PrefetchScalarGridSpec(
            num_scalar_prefetch=1,
            grid=grid,
            in_specs=[
                pl.BlockSpec((TM, D_in), lambda m, n, u: (m, 0)),
                pl.BlockSpec((TN, D_in), lambda m, n, u: (n, 0)),
                pl.BlockSpec((1, TN), lambda m, n, u: (0, n)),
                pl.BlockSpec((34, D_in), lambda m, n, u: (0, 0)),
                pl.BlockSpec((2, D_in), lambda m, n, u: (0, 0)),
                pl.BlockSpec((1, 1, D_in), lambda m, n, u: (u[0], 0, 0)),
                pl.BlockSpec((TN, 2 * _R), lambda m, n, u: (n, 0)),
            ],
            out_specs=pl.BlockSpec((TM, TN), lambda m, n, u: (m, n)),
            scratch_shapes=[pltpu.VMEM((TM, 2 * _R), jnp.float32)],
        ),
        out_shape=jax.ShapeDtypeStruct((M, D_out), jnp.float32),
        compiler_params=pltpu.CompilerParams(
            dimension_semantics=("parallel", "arbitrary"),
        ),
    )(uid, x2, W, b2, Wcat, W_route_user, user_emb, Bcat)
    return out.reshape(Bb, S, D_out)


# speedup vs baseline: 1.0638x; 1.0493x over previous
import jax
import jax.numpy as jnp
from jax.experimental import pallas as pl
from jax.experimental.pallas import tpu as pltpu

_LORA_SCALE = 0.2 * (32 / 16)
_R = 16


def _dot_t(a, b):
    return jax.lax.dot_general(a, b, (((1,), (1,)), ((), ())),
                               preferred_element_type=jnp.float32)


def _fused_body(uid_ref, x_ref, w_ref, b_ref, wcat_ref, wru_ref, uemb_ref,
                bcat_ref, o_ref, v_ref):
    n = pl.program_id(1)

    @pl.when(n == 0)
    def _router():
        r = _dot_t(x_ref[...], wcat_ref[...])
        lu = _dot_t(uemb_ref[0], wru_ref[...])
        pi0 = jax.nn.sigmoid(r[:, 32:33] - r[:, 33:34])
        pu0 = jax.nn.sigmoid(lu[:, 0:1] - lu[:, 1:2])
        pm0 = jax.nn.sigmoid(2.0 * (pi0 + pu0) - 2.0)
        v_ref[:, :_R] = r[:, :_R] * (_LORA_SCALE * pm0)
        v_ref[:, _R:] = r[:, _R:2 * _R] * (_LORA_SCALE * (1.0 - pm0))

    acc = _dot_t(x_ref[...], w_ref[...])
    acc += _dot_t(v_ref[...], bcat_ref[...])
    o_ref[...] = acc + b_ref[...]


def kernel(x, user_id, W, b, W_route_in, W_route_user, user_emb, A0, B0, A1, B1):
    Bb, S, D_in = x.shape
    D_out = W.shape[0]
    M = Bb * S
    TM, TN = 1024, 512
    x2 = x.reshape(M, D_in)
    b2 = b.reshape(1, D_out)
    uid = (user_id[0].astype(jnp.int32) - 1)[None]
    Wcat = jnp.concatenate([A0, A1, W_route_in], axis=0)
    user_emb = user_emb.reshape(user_emb.shape[0], 1, D_in)
    Bcat = jnp.concatenate([B0, B1], axis=1)

    grid = (M // TM, D_out // TN)
    out = pl.pallas_call(
        _fused_body,
        grid_spec=pltpu.PrefetchScalarGridSpec(
            num_scalar_prefetch=1,
            grid=grid,
            in_specs=[
                pl.BlockSpec((TM, D_in), lambda m, n, u: (m, 0)),
                pl.BlockSpec((TN, D_in), lambda m, n, u: (n, 0)),
                pl.BlockSpec((1, TN), lambda m, n, u: (0, n)),
                pl.BlockSpec((34, D_in), lambda m, n, u: (0, 0)),
                pl.BlockSpec((2, D_in), lambda m, n, u: (0, 0)),
                pl.BlockSpec((1, 1, D_in), lambda m, n, u: (u[0], 0, 0)),
                pl.BlockSpec((TN, 2 * _R), lambda m, n, u: (n, 0)),
            ],
            out_specs=pl.BlockSpec((TM, TN), lambda m, n, u: (m, n)),
            scratch_shapes=[pltpu.VMEM((TM, 2 * _R), jnp.float32)],
        ),
        out_shape=jax.ShapeDtypeStruct((M, D_out), jnp.float32),
        compiler_params=pltpu.CompilerParams(
            dimension_semantics=("arbitrary", "arbitrary"),
        ),
    )(uid, x2, W, b2, Wcat, W_route_user, user_emb, Bcat)
    return out.reshape(Bb, S, D_out)


# speedup vs baseline: 1.0676x; 1.0035x over previous
import jax
import jax.numpy as jnp
from jax.experimental import pallas as pl
from jax.experimental.pallas import tpu as pltpu

_LORA_SCALE = 0.2 * (32 / 16)
_R = 16


def _dot_t(a, b):
    return jax.lax.dot_general(a, b, (((1,), (1,)), ((), ())),
                               preferred_element_type=jnp.float32)


def _fused_body(uid_ref, x_ref, w_ref, wcat_ref, wru_ref, uemb_ref,
                bcat_ref, o_ref, v_ref):
    n = pl.program_id(1)

    @pl.when(n == 0)
    def _router():
        r = _dot_t(x_ref[...], wcat_ref[...])
        lu = _dot_t(uemb_ref[0], wru_ref[...])
        pi0 = jax.nn.sigmoid(r[:, 32:33] - r[:, 33:34])
        pu0 = jax.nn.sigmoid(lu[:, 0:1] - lu[:, 1:2])
        pm0 = jax.nn.sigmoid(2.0 * (pi0 + pu0) - 2.0)
        tm = v_ref.shape[0]
        v_ref[:, :_R] = r[:, :_R] * (_LORA_SCALE * pm0)
        v_ref[:, _R:2 * _R] = r[:, _R:2 * _R] * (_LORA_SCALE * (1.0 - pm0))
        v_ref[:, 2 * _R:] = jnp.ones((tm, 1), jnp.float32)

    acc = _dot_t(x_ref[...], w_ref[...])
    acc += _dot_t(v_ref[...], bcat_ref[...])
    o_ref[...] = acc


def kernel(x, user_id, W, b, W_route_in, W_route_user, user_emb, A0, B0, A1, B1):
    Bb, S, D_in = x.shape
    D_out = W.shape[0]
    M = Bb * S
    TM, TN = 1024, 512
    x2 = x.reshape(M, D_in)
    uid = (user_id[0].astype(jnp.int32) - 1)[None]
    Wcat = jnp.concatenate([A0, A1, W_route_in], axis=0)
    user_emb = user_emb.reshape(user_emb.shape[0], 1, D_in)
    Bcat = jnp.concatenate([B0, B1, b[:, None]], axis=1)

    grid = (M // TM, D_out // TN)
    out = pl.pallas_call(
        _fused_body,
        grid_spec=pltpu.PrefetchScalarGridSpec(
            num_scalar_prefetch=1,
            grid=grid,
            in_specs=[
                pl.BlockSpec((TM, D_in), lambda m, n, u: (m, 0)),
                pl.BlockSpec((TN, D_in), lambda m, n, u: (n, 0)),
                pl.BlockSpec((34, D_in), lambda m, n, u: (0, 0)),
                pl.BlockSpec((2, D_in), lambda m, n, u: (0, 0)),
                pl.BlockSpec((1, 1, D_in), lambda m, n, u: (u[0], 0, 0)),
                pl.BlockSpec((TN, 2 * _R + 1), lambda m, n, u: (n, 0)),
            ],
            out_specs=pl.BlockSpec((TM, TN), lambda m, n, u: (m, n)),
            scratch_shapes=[pltpu.VMEM((TM, 2 * _R + 1), jnp.float32)],
        ),
        out_shape=jax.ShapeDtypeStruct((M, D_out), jnp.float32),
        compiler_params=pltpu.CompilerParams(
            dimension_semantics=("arbitrary", "arbitrary"),
        ),
    )(uid, x2, W, Wcat, W_route_user, user_emb, Bcat)
    return out.reshape(Bb, S, D_out)
